# baseline (speedup 1.0000x reference)
import functools

import jax
import jax.numpy as jnp
from jax import lax
from jax.experimental import pallas as pl
from jax.experimental.pallas import tpu as pltpu
from jax.experimental.pallas import tpu_sc as plsc

VOCAB = 100000
D = 64
B = 1024
CTX = 20

NC = 2
NS = 16
NW = NC * NS
BPW = B // NW
LANES = 16
KV = D // LANES
IPW = BPW * CTX
NCHUNK = IPW // 128
PAIRW = 2 * D

VBLK = 2048
NVBLK = (VOCAB + VBLK - 1) // VBLK
TAIL = VOCAB - (NVBLK - 1) * VBLK
VPAD = NVBLK * VBLK
NBUF = 4


@functools.cache
def _make_gather_mean():
    mesh = plsc.VectorSubcoreMesh(core_axis_name="c", subcore_axis_name="s")

    @functools.partial(
        pl.kernel,
        mesh=mesh,
        out_type=jax.ShapeDtypeStruct((NW, D, BPW), jnp.float32),
        scratch_types=[
            pltpu.VMEM((NCHUNK, 128), jnp.int32),
            pltpu.VMEM((NCHUNK, 128), jnp.int32),
            pltpu.VMEM((NCHUNK, 128, PAIRW), jnp.float32),
            pltpu.VMEM((D, BPW), jnp.float32),
            pltpu.SemaphoreType.DMA,
        ],
        compiler_params=pltpu.CompilerParams(
            needs_layout_passes=False, use_tc_tiling_on_sc=True),
    )
    def _gather_mean(idx_hbm, tbl2_hbm, out_hbm, idx_v, offs_v, rows_v, acc_v, sem):
        wid = lax.axis_index("s") * NC + lax.axis_index("c")
        pltpu.sync_copy(idx_hbm.at[wid], idx_v)
        for q in range(NCHUNK):
            for t in range(128 // LANES):
                sl = pl.ds(t * LANES, LANES)
                v = idx_v[q, sl]
                offs_v[q, sl] = lax.shift_left(
                    jnp.bitwise_and(v, jnp.int32(1)), 6)
                idx_v[q, sl] = lax.shift_right_logical(v, 1)
        copies = []
        for q in range(NCHUNK):
            copies.append(
                pltpu.async_copy(tbl2_hbm.at[idx_v.at[q]], rows_v.at[q], sem))
        for c in copies:
            c.wait()

        def d_body(d, carry):
            for half in range(2):
                acc = jnp.zeros((LANES,), jnp.float32)
                for j in range(CTX):
                    pj = j * BPW + half * LANES
                    cj, lj = pj // 128, pj % 128
                    cjv = jnp.full((LANES,), cj, jnp.int32)
                    ljv = lax.iota(jnp.int32, LANES) + lj
                    colv = offs_v[cj, pl.ds(lj, LANES)] + d
                    acc = acc + plsc.load_gather(rows_v, [cjv, ljv, colv])
                acc_v[d, pl.ds(half * LANES, LANES)] = acc * (1.0 / CTX)
            return carry

        lax.fori_loop(0, D, d_body, 0)
        pltpu.sync_copy(acc_v, out_hbm.at[wid])

    return _gather_mean


def _mm_body(mean_ref, wt_ref, b_ref, out_hbm, acc_ref, tail_ref, sems, tail_sem):
    i = pl.program_id(0)
    slot = lax.rem(i, NBUF)

    @pl.when(i >= NBUF)
    def _():
        prev = i - NBUF
        pltpu.make_async_copy(
            acc_ref.at[slot],
            out_hbm.at[pl.ds(prev * VBLK, VBLK), :],
            sems.at[slot],
        ).wait()

    waug = jnp.concatenate([wt_ref[...], b_ref[0]], axis=0)
    maug = jnp.concatenate(
        [mean_ref[...], jnp.ones((1, B), jnp.float32)], axis=0)
    blk = lax.dot_general(
        waug, maug,
        dimension_numbers=(((0,), (0,)), ((), ())),
        preferred_element_type=jnp.float32,
    )

    @pl.when(i < NVBLK - 1)
    def _():
        acc_ref[slot] = blk
        pltpu.make_async_copy(
            acc_ref.at[slot],
            out_hbm.at[pl.ds(i * VBLK, VBLK), :],
            sems.at[slot],
        ).start()

    @pl.when(i == NVBLK - 1)
    def _():
        tail_ref[...] = blk[:TAIL]
        pltpu.make_async_copy(
            tail_ref,
            out_hbm.at[pl.ds((NVBLK - 1) * VBLK, TAIL), :],
            tail_sem,
        ).start()
        for k in range(NBUF - 1):
            step = NVBLK - NBUF + k
            pltpu.make_async_copy(
                acc_ref.at[step % NBUF],
                out_hbm.at[pl.ds(step * VBLK, VBLK), :],
                sems.at[step % NBUF],
            ).wait()
        pltpu.make_async_copy(
            tail_ref,
            out_hbm.at[pl.ds((NVBLK - 1) * VBLK, TAIL), :],
            tail_sem,
        ).wait()


@functools.cache
def _make_matmul():
    return pl.pallas_call(
        _mm_body,
        grid=(NVBLK,),
        in_specs=[
            pl.BlockSpec((D, B), lambda i: (0, 0)),
            pl.BlockSpec((D, VBLK), lambda i: (0, i)),
            pl.BlockSpec((1, 1, VBLK), lambda i: (i, 0, 0)),
        ],
        out_specs=pl.BlockSpec(memory_space=pltpu.HBM),
        out_shape=jax.ShapeDtypeStruct((VOCAB, B), jnp.float32),
        scratch_shapes=[
            pltpu.VMEM((NBUF, VBLK, B), jnp.float32),
            pltpu.VMEM((TAIL, B), jnp.float32),
            pltpu.SemaphoreType.DMA((NBUF,)),
            pltpu.SemaphoreType.DMA,
        ],
        compiler_params=pltpu.CompilerParams(
            dimension_semantics=("arbitrary",),
        ),
    )


def kernel(context_window, emb_table, W, b):
    idx = (context_window.astype(jnp.int32)
           .reshape(NW, BPW, CTX).transpose(0, 2, 1).reshape(NW, NCHUNK, 128))
    tbl2 = emb_table.reshape(VOCAB // 2, PAIRW)
    mean3 = _make_gather_mean()(idx, tbl2)
    mean_t = mean3.transpose(1, 0, 2).reshape(D, B)
    b_pad = jnp.pad(b, (0, VPAD - VOCAB)).reshape(NVBLK, 1, VBLK)
    logits_t = _make_matmul()(mean_t, W.T, b_pad)
    return logits_t.T


# speedup vs baseline: 1.1540x; 1.1540x over previous
import functools

import jax
import jax.numpy as jnp
from jax import lax
from jax.experimental import pallas as pl
from jax.experimental.pallas import tpu as pltpu
from jax.experimental.pallas import tpu_sc as plsc

VOCAB = 100000
D = 64
B = 1024
CTX = 20

NC = 2
NS = 16
NW = NC * NS
BPW = B // NW
LANES = 16
KV = D // LANES
IPW = BPW * CTX
NCHUNK = IPW // 128
PAIRW = 2 * D

VBLK = 2048
NVBLK = (VOCAB + VBLK - 1) // VBLK
TAIL = VOCAB - (NVBLK - 1) * VBLK
VPAD = NVBLK * VBLK
NBUF = 4


@functools.cache
def _make_gather_mean():
    mesh = plsc.VectorSubcoreMesh(core_axis_name="c", subcore_axis_name="s")

    @functools.partial(
        pl.kernel,
        mesh=mesh,
        out_type=jax.ShapeDtypeStruct((NW, D, BPW), jnp.float32),
        scratch_types=[
            pltpu.VMEM((NCHUNK, 128), jnp.int32),
            pltpu.VMEM((NCHUNK, 128), jnp.int32),
            pltpu.VMEM((NCHUNK, 128, PAIRW), jnp.float32),
            pltpu.VMEM((D, BPW), jnp.float32),
            pltpu.SemaphoreType.DMA,
        ],
        compiler_params=pltpu.CompilerParams(
            needs_layout_passes=False, use_tc_tiling_on_sc=True),
    )
    def _gather_mean(idx_hbm, tbl2_hbm, out_hbm, idx_v, offs_v, rows_v, acc_v, sem):
        wid = lax.axis_index("s") * NC + lax.axis_index("c")
        pltpu.sync_copy(idx_hbm.at[wid], idx_v)
        for q in range(NCHUNK):
            for t in range(128 // LANES):
                sl = pl.ds(t * LANES, LANES)
                v = idx_v[q, sl]
                offs_v[q, sl] = lax.shift_left(
                    jnp.bitwise_and(lax.shift_right_logical(v, 12),
                                    jnp.int32(1)), 6)
                idx_v[q, sl] = (
                    lax.shift_left(lax.shift_right_logical(v, 13), 12)
                    + jnp.bitwise_and(v, jnp.int32(4095)))
        copies = []
        for q in range(NCHUNK):
            copies.append(
                pltpu.async_copy(tbl2_hbm.at[idx_v.at[q]], rows_v.at[q], sem))
        for c in copies:
            c.wait()

        def d_body(d, carry):
            for half in range(2):
                acc = jnp.zeros((LANES,), jnp.float32)
                for j in range(CTX):
                    pj = j * BPW + half * LANES
                    cj, lj = pj // 128, pj % 128
                    cjv = jnp.full((LANES,), cj, jnp.int32)
                    ljv = lax.iota(jnp.int32, LANES) + lj
                    colv = offs_v[cj, pl.ds(lj, LANES)] + d
                    acc = acc + plsc.load_gather(rows_v, [cjv, ljv, colv])
                acc_v[d, pl.ds(half * LANES, LANES)] = acc * (1.0 / CTX)
            return carry

        lax.fori_loop(0, D, d_body, 0)
        pltpu.sync_copy(acc_v, out_hbm.at[wid])

    return _gather_mean


RPV = 8192
RPO = RPV // 2
NRP = (VOCAB + RPV - 1) // RPV
NPAIR = NRP * RPO


def _repack_body(ta_ref, tb_ref, out_ref):
    t0 = jnp.transpose(ta_ref[...], (1, 0))
    t1 = jnp.transpose(tb_ref[...], (1, 0))
    out_ref[...] = jnp.concatenate([t0, t1], axis=1)


@functools.cache
def _make_repack():
    return pl.pallas_call(
        _repack_body,
        grid=(NRP,),
        in_specs=[
            pl.BlockSpec((D, RPO), lambda i: (0, 2 * i)),
            pl.BlockSpec(
                (D, RPO),
                lambda i: (0, jnp.minimum(2 * i + 1, 2 * (NRP - 1)))),
        ],
        out_specs=pl.BlockSpec((RPO, PAIRW), lambda i: (i, 0)),
        out_shape=jax.ShapeDtypeStruct((NPAIR, PAIRW), jnp.float32),
        compiler_params=pltpu.CompilerParams(
            dimension_semantics=("arbitrary",),
        ),
    )


def _mm_body(mean_ref, wt_ref, b_ref, out_hbm, acc_ref, tail_ref, sems, tail_sem):
    i = pl.program_id(0)
    slot = lax.rem(i, NBUF)

    @pl.when(i >= NBUF)
    def _():
        prev = i - NBUF
        pltpu.make_async_copy(
            acc_ref.at[slot],
            out_hbm.at[pl.ds(prev * VBLK, VBLK), :],
            sems.at[slot],
        ).wait()

    waug = jnp.concatenate([wt_ref[...], b_ref[0]], axis=0)
    maug = jnp.concatenate(
        [mean_ref[...], jnp.ones((1, B), jnp.float32)], axis=0)
    blk = lax.dot_general(
        waug, maug,
        dimension_numbers=(((0,), (0,)), ((), ())),
        preferred_element_type=jnp.float32,
    )

    @pl.when(i < NVBLK - 1)
    def _():
        acc_ref[slot] = blk
        pltpu.make_async_copy(
            acc_ref.at[slot],
            out_hbm.at[pl.ds(i * VBLK, VBLK), :],
            sems.at[slot],
        ).start()

    @pl.when(i == NVBLK - 1)
    def _():
        tail_ref[...] = blk[:TAIL]
        pltpu.make_async_copy(
            tail_ref,
            out_hbm.at[pl.ds((NVBLK - 1) * VBLK, TAIL), :],
            tail_sem,
        ).start()
        for k in range(NBUF - 1):
            step = NVBLK - NBUF + k
            pltpu.make_async_copy(
                acc_ref.at[step % NBUF],
                out_hbm.at[pl.ds(step * VBLK, VBLK), :],
                sems.at[step % NBUF],
            ).wait()
        pltpu.make_async_copy(
            tail_ref,
            out_hbm.at[pl.ds((NVBLK - 1) * VBLK, TAIL), :],
            tail_sem,
        ).wait()


@functools.cache
def _make_matmul():
    return pl.pallas_call(
        _mm_body,
        grid=(NVBLK,),
        in_specs=[
            pl.BlockSpec((D, B), lambda i: (0, 0)),
            pl.BlockSpec((D, VBLK), lambda i: (0, i)),
            pl.BlockSpec((1, 1, VBLK), lambda i: (i, 0, 0)),
        ],
        out_specs=pl.BlockSpec(memory_space=pltpu.HBM),
        out_shape=jax.ShapeDtypeStruct((VOCAB, B), jnp.float32),
        scratch_shapes=[
            pltpu.VMEM((NBUF, VBLK, B), jnp.float32),
            pltpu.VMEM((TAIL, B), jnp.float32),
            pltpu.SemaphoreType.DMA((NBUF,)),
            pltpu.SemaphoreType.DMA,
        ],
        compiler_params=pltpu.CompilerParams(
            dimension_semantics=("arbitrary",),
        ),
    )


def kernel(context_window, emb_table, W, b):
    idx = (context_window.astype(jnp.int32)
           .reshape(NW, BPW, CTX).transpose(0, 2, 1).reshape(NW, NCHUNK, 128))
    tbl2 = _make_repack()(emb_table.T, emb_table.T)
    mean3 = _make_gather_mean()(idx, tbl2)
    mean_t = mean3.transpose(1, 0, 2).reshape(D, B)
    b_pad = jnp.pad(b, (0, VPAD - VOCAB)).reshape(NVBLK, 1, VBLK)
    logits_t = _make_matmul()(mean_t, W.T, b_pad)
    return logits_t.T
